# Initial kernel scaffold; baseline (speedup 1.0000x reference)
import jax
import jax.numpy as jnp
from jax.experimental import pallas as pl


def kernel(x, w_tod, w_dow, w_dom, w_doy):
    raise NotImplementedError("write your pallas kernel here")


# speedup vs baseline: 3.6997x; 3.6997x over previous
import jax
import jax.numpy as jnp
from jax import lax
from jax.experimental import pallas as pl
from jax.experimental.pallas import tpu as pltpu
from jax.experimental.pallas import tpu_sc as plsc

TOD, DOW, DOM, DOY = 288, 7, 31, 366
D = 64
N_TOK = 4096 * 200
NC, NS = 2, 16
NW = NC * NS
TOK_PER_W = N_TOK // NW
T = 256
N_CHUNK = TOK_PER_W // T
G = T // 16
R_DD = DOW * DOM
ROWS = TOD + R_DD + DOY


def _sc_body(x_hbm, tab_hbm, out_hbm, tab_v, x_v, out_v):
    wid = lax.axis_index("s") * NC + lax.axis_index("c")
    pltpu.sync_copy(tab_hbm, tab_v)
    iota = lax.iota(jnp.int32, 16)

    def chunk_body(ci, carry):
        base = wid * TOK_PER_W + ci * T
        pltpu.sync_copy(x_hbm.at[pl.ds(base * 4, T * 4)], x_v)

        def group_body(g, carry2):
            tvec = g * 16 + iota
            xpos = tvec * 4
            x0 = plsc.load_gather(x_v, [xpos])
            x1 = plsc.load_gather(x_v, [xpos + 1])
            x2 = plsc.load_gather(x_v, [xpos + 2])
            x3 = plsc.load_gather(x_v, [xpos + 3])
            i_tod = ((x0 + 0.5) * float(TOD)).astype(jnp.int32)
            i_dow = ((x1 + 0.5) * float(DOW)).astype(jnp.int32)
            i_dom = ((x2 + 0.5) * float(DOM)).astype(jnp.int32)
            i_doy = ((x3 + 0.5) * float(DOY)).astype(jnp.int32)
            b0 = i_tod * D
            b1 = (TOD + i_dow * DOM + i_dom) * D
            b2 = ((TOD + R_DD) + i_doy) * D
            ob = tvec * D
            for col in range(D):
                v = (plsc.load_gather(tab_v, [b0 + col])
                     + plsc.load_gather(tab_v, [b1 + col])
                     + plsc.load_gather(tab_v, [b2 + col]))
                plsc.store_scatter(out_v, [ob + col], v)
            return carry2

        lax.fori_loop(0, G, group_body, 0)
        pltpu.sync_copy(out_v, out_hbm.at[pl.ds(base * D, T * D)])
        return carry

    lax.fori_loop(0, N_CHUNK, chunk_body, 0)


def kernel(x, w_tod, w_dow, w_dom, w_doy):
    w_dd = (w_dow[:, None, :] + w_dom[None, :, :]).reshape(R_DD, D)
    tab = jnp.concatenate([w_tod, w_dd, w_doy], axis=0).reshape(-1)
    x_flat = x.reshape(-1)
    mesh = plsc.VectorSubcoreMesh(core_axis_name="c", subcore_axis_name="s")
    out = pl.kernel(
        _sc_body,
        out_type=jax.ShapeDtypeStruct((N_TOK * D,), jnp.float32),
        mesh=mesh,
        scratch_types=[
            pltpu.VMEM((ROWS * D,), jnp.float32),
            pltpu.VMEM((T * 4,), jnp.float32),
            pltpu.VMEM((T * D,), jnp.float32),
        ],
        compiler_params=pltpu.CompilerParams(needs_layout_passes=False),
    )(x_flat, tab)
    return out.reshape(4096, 200, D)


# speedup vs baseline: 9.7249x; 2.6286x over previous
import jax
import jax.numpy as jnp
from jax import lax
from jax.experimental import pallas as pl
from jax.experimental.pallas import tpu as pltpu
from jax.experimental.pallas import tpu_sc as plsc

TOD, DOW, DOM, DOY = 288, 7, 31, 366
D = 64
B = 4096
S = 200
N_TOK = B * S
NC, NS = 2, 16
NW = NC * NS
ROWS_PER_W = B // NW
R_DD = DOW * DOM
ROWS = TOD + R_DD + DOY


def _sc_body(x_hbm, tab_hbm, out_hbm, tab_v, x_v, out_v):
    wid = lax.axis_index("s") * NC + lax.axis_index("c")
    pltpu.sync_copy(tab_hbm, tab_v)

    iota = lax.iota(jnp.int32, 16)
    lane = iota & 3
    quad = iota >> 2
    scale = jnp.where(lane == 0, float(TOD),
                      jnp.where(lane == 1, float(DOW),
                                jnp.where(lane == 2, float(DOM), float(DOY))))
    mul = jnp.where(lane == 1, DOM * D, D)
    off = jnp.where(lane == 1, TOD * D,
                    jnp.where(lane == 3, (TOD + R_DD) * D, 0))

    def chunk_body(ci, carry):
        b = wid * ROWS_PER_W + ci
        pltpu.sync_copy(x_hbm.at[b], x_v)

        @plsc.parallel_loop(0, S // 4, unroll=4)
        def quad_body(g):
            xv = plsc.load_gather(x_v, [4 * g + quad, lane])
            cvec = ((xv + 0.5) * scale).astype(jnp.int32) * mul + off
            for k in range(4):
                r0 = cvec[4 * k]
                r1 = cvec[4 * k + 1] + cvec[4 * k + 2]
                r2 = cvec[4 * k + 3]
                t = 4 * g + k
                for c in range(0, D, 16):
                    v = (tab_v[pl.ds(r0 + c, 16)]
                         + tab_v[pl.ds(r1 + c, 16)]
                         + tab_v[pl.ds(r2 + c, 16)])
                    out_v[t, pl.ds(c, 16)] = v

        pltpu.sync_copy(out_v, out_hbm.at[b])
        return carry

    lax.fori_loop(0, ROWS_PER_W, chunk_body, 0)


def kernel(x, w_tod, w_dow, w_dom, w_doy):
    w_dd = (w_dow[:, None, :] + w_dom[None, :, :]).reshape(R_DD, D)
    tab = jnp.concatenate([w_tod, w_dd, w_doy], axis=0).reshape(-1)
    mesh = plsc.VectorSubcoreMesh(core_axis_name="c", subcore_axis_name="s")
    out = pl.kernel(
        _sc_body,
        out_type=jax.ShapeDtypeStruct((B, S, D), jnp.float32),
        mesh=mesh,
        scratch_types=[
            pltpu.VMEM((ROWS * D,), jnp.float32),
            pltpu.VMEM((S, 4), jnp.float32),
            pltpu.VMEM((S, D), jnp.float32),
        ],
        compiler_params=pltpu.CompilerParams(needs_layout_passes=False),
    )(x, tab)
    return out
